# Initial kernel scaffold; baseline (speedup 1.0000x reference)
import jax
import jax.numpy as jnp
from jax.experimental import pallas as pl


def kernel(x, edge_index, batch, gcn_W, gcn_b, gat_W, att_src, att_dst, gat_b, out_W, out_b):
    raise NotImplementedError("write your pallas kernel here")


# speedup vs baseline: 22.9173x; 22.9173x over previous
import jax
import jax.numpy as jnp
from jax import lax
from jax.experimental import pallas as pl
from jax.experimental.pallas import tpu as pltpu
from jax.experimental.pallas import tpu_sc as plsc

N = 10000
NP = 10240
E = 320000
D = 128
NG = 16

NC = 2
NS = 16
L = 16
NW = NC * NS
EPT = E // NW
C = 80
NCH = EPT // C
BLK = 25
NBLK = NCH // BLK
RPT = NP // NS

RB = 1024
NB = NP // RB

_f32 = jnp.float32
_i32 = jnp.int32


def _mesh():
    return plsc.VectorSubcoreMesh(
        core_axis_name="c", subcore_axis_name="s", num_cores=NC, num_subcores=NS
    )


def _sc_deg_body(dstr, zN, o1, out, dst_v, obuf, deg_sp):
    c = lax.axis_index("c")
    s = lax.axis_index("s")
    wid = s * NC + c
    rows = pl.ds(s * RPT, RPT)
    pltpu.sync_copy(dstr.at[wid], dst_v)
    pltpu.sync_copy(zN.at[rows], deg_sp.at[rows])
    pltpu.sync_copy(o1, obuf)
    plsc.subcore_barrier()

    def body(j, carry):
        pltpu.sync_copy(obuf, deg_sp.at[dst_v.at[j]], add=True)
        return carry

    lax.fori_loop(0, NCH, body, 0)
    plsc.subcore_barrier()
    pltpu.sync_copy(deg_sp.at[rows], out.at[c, rows])


def _sc_deg(dstr, zN, o1):
    kern = pl.kernel(
        _sc_deg_body,
        out_type=jax.ShapeDtypeStruct((NC, NP), _f32),
        mesh=_mesh(),
        scratch_types=[
            pltpu.VMEM((NCH, C), _i32),
            pltpu.VMEM((C,), _f32),
            pltpu.VMEM_SHARED((NP,), _f32),
        ],
    )
    return kern(dstr, zN, o1)


def _sc_spmm_body(y, srcr, dstr, z128, out, src_v, dst_v, buf, acc_sp):
    c = lax.axis_index("c")
    s = lax.axis_index("s")
    wid = s * NC + c
    rows = pl.ds(s * RPT, RPT)
    pltpu.sync_copy(srcr.at[wid], src_v)
    pltpu.sync_copy(dstr.at[wid], dst_v)
    pltpu.sync_copy(z128.at[rows], acc_sp.at[rows])
    plsc.subcore_barrier()

    def body(j, carry):
        pltpu.sync_copy(y.at[src_v.at[j]], buf)
        pltpu.sync_copy(buf, acc_sp.at[dst_v.at[j]], add=True)
        return carry

    lax.fori_loop(0, NCH, body, 0)
    plsc.subcore_barrier()
    pltpu.sync_copy(acc_sp.at[rows], out.at[c, rows])


def _sc_spmm(y, srcr, dstr, z128):
    kern = pl.kernel(
        _sc_spmm_body,
        out_type=jax.ShapeDtypeStruct((NC, NP, D), _f32),
        mesh=_mesh(),
        scratch_types=[
            pltpu.VMEM((NCH, C), _i32),
            pltpu.VMEM((NCH, C), _i32),
            pltpu.VMEM((C, D), _f32),
            pltpu.VMEM_SHARED((NP, D), _f32),
        ],
    )
    return kern(y, srcr, dstr, z128)


def _sc_gat_body(hw, a_s, a_d, amax, srcr, dstr, z128, zN, g_out, z_out,
                 src_v, dst_v, asb, adb, amx, pbuf, buf, g_sp, z1_sp):
    c = lax.axis_index("c")
    s = lax.axis_index("s")
    wid = s * NC + c
    rows = pl.ds(s * RPT, RPT)
    pltpu.sync_copy(amax, amx)
    pltpu.sync_copy(z128.at[rows], g_sp.at[rows])
    pltpu.sync_copy(zN.at[rows], z1_sp.at[rows])
    plsc.subcore_barrier()

    def blk_body(b, carry):
        pltpu.sync_copy(srcr.at[wid, b], src_v)
        pltpu.sync_copy(dstr.at[wid, b], dst_v)

        def body(k, carry2):
            pltpu.sync_copy(hw.at[src_v.at[k]], buf)
            pltpu.sync_copy(a_s.at[src_v.at[k]], asb)
            pltpu.sync_copy(a_d.at[dst_v.at[k]], adb)
            amx_v = amx[pl.ds(0, L)]

            def scale(g, carry3):
                sl = pl.ds(g * L, L)
                av = asb[sl]
                bv = adb[sl]
                e = av + bv
                e = jnp.where(e >= 0.0, e, 0.2 * e)
                mh = amx_v + bv
                mh = jnp.where(mh >= 0.0, mh, 0.2 * mh)
                p = jnp.exp(e - mh)
                pbuf[sl] = p
                for r in range(L):
                    lrow = g * L + r
                    ps = p[r]
                    for v8 in range(D // L):
                        fsl = pl.ds(v8 * L, L)
                        buf[lrow, fsl] = buf[lrow, fsl] * ps
                return carry3

            lax.fori_loop(0, C // L, scale, 0)
            pltpu.sync_copy(buf, g_sp.at[dst_v.at[k]], add=True)
            pltpu.sync_copy(pbuf, z1_sp.at[dst_v.at[k]], add=True)
            return carry2

        return lax.fori_loop(0, BLK, body, carry)

    lax.fori_loop(0, NBLK, blk_body, 0)
    plsc.subcore_barrier()
    pltpu.sync_copy(g_sp.at[rows], g_out.at[c, rows])
    pltpu.sync_copy(z1_sp.at[rows], z_out.at[c, rows])


def _sc_gat(hw, a_s, a_d, amax, srcr, dstr, z128, zN):
    kern = pl.kernel(
        _sc_gat_body,
        out_type=(
            jax.ShapeDtypeStruct((NC, NP, D), _f32),
            jax.ShapeDtypeStruct((NC, NP), _f32),
        ),
        mesh=_mesh(),
        scratch_types=[
            pltpu.VMEM((BLK, C), _i32),
            pltpu.VMEM((BLK, C), _i32),
            pltpu.VMEM((C,), _f32),
            pltpu.VMEM((C,), _f32),
            pltpu.VMEM((L,), _f32),
            pltpu.VMEM((C,), _f32),
            pltpu.VMEM((C, D), _f32),
            pltpu.VMEM_SHARED((NP, D), _f32),
            pltpu.VMEM_SHARED((NP,), _f32),
        ],
    )
    return kern(hw, a_s, a_d, amax, srcr, dstr, z128, zN)


def _tc1_body(x_ref, w_ref, deg_ref, y_ref, dinv_ref):
    deg = deg_ref[0] + deg_ref[1] + 1.0
    dinv = lax.rsqrt(deg)
    xw = jnp.dot(x_ref[...], w_ref[...], preferred_element_type=_f32)
    y_ref[...] = xw * dinv
    dinv_ref[...] = dinv


def _tc1(x, gcn_W, deg_p):
    return pl.pallas_call(
        _tc1_body,
        grid=(NB,),
        in_specs=[
            pl.BlockSpec((RB, D), lambda i: (i, 0)),
            pl.BlockSpec((D, D), lambda i: (0, 0)),
            pl.BlockSpec((NC, RB, 1), lambda i: (0, i, 0)),
        ],
        out_specs=[
            pl.BlockSpec((RB, D), lambda i: (i, 0)),
            pl.BlockSpec((RB, 1), lambda i: (i, 0)),
        ],
        out_shape=[
            jax.ShapeDtypeStruct((NP, D), _f32),
            jax.ShapeDtypeStruct((NP, 1), _f32),
        ],
    )(x, gcn_W, deg_p)


def _tc2_body(acc_ref, y_ref, dinv_ref, gcnb_ref, gatW_ref, atts_ref, attd_ref,
              hw_ref, as_ref, ad_ref, amax_ref):
    i = pl.program_id(0)
    acc = acc_ref[0] + acc_ref[1] + y_ref[...]
    h = jnp.maximum(acc * dinv_ref[...] + gcnb_ref[...], 0.0)
    hw = jnp.dot(h, gatW_ref[...], preferred_element_type=_f32)
    hw_ref[...] = hw
    a_s = jnp.dot(hw, atts_ref[...], preferred_element_type=_f32)
    a_d = jnp.dot(hw, attd_ref[...], preferred_element_type=_f32)
    as_ref[...] = a_s
    ad_ref[...] = a_d
    m = jnp.max(a_s)

    @pl.when(i == 0)
    def _():
        amax_ref[0, 0] = m

    @pl.when(i > 0)
    def _():
        amax_ref[0, 0] = jnp.maximum(amax_ref[0, 0], m)


def _tc2(acc_p, y, dinv, gcn_b, gat_W, atts, attd):
    return pl.pallas_call(
        _tc2_body,
        grid=(NB,),
        in_specs=[
            pl.BlockSpec((NC, RB, D), lambda i: (0, i, 0)),
            pl.BlockSpec((RB, D), lambda i: (i, 0)),
            pl.BlockSpec((RB, 1), lambda i: (i, 0)),
            pl.BlockSpec((1, D), lambda i: (0, 0)),
            pl.BlockSpec((D, D), lambda i: (0, 0)),
            pl.BlockSpec((D, 1), lambda i: (0, 0)),
            pl.BlockSpec((D, 1), lambda i: (0, 0)),
        ],
        out_specs=[
            pl.BlockSpec((RB, D), lambda i: (i, 0)),
            pl.BlockSpec((RB, 1), lambda i: (i, 0)),
            pl.BlockSpec((RB, 1), lambda i: (i, 0)),
            pl.BlockSpec(memory_space=pltpu.SMEM),
        ],
        out_shape=[
            jax.ShapeDtypeStruct((NP, D), _f32),
            jax.ShapeDtypeStruct((NP, 1), _f32),
            jax.ShapeDtypeStruct((NP, 1), _f32),
            jax.ShapeDtypeStruct((1, 1), _f32),
        ],
    )(acc_p, y, dinv, gcn_b, gat_W, atts, attd)


def _tc3_body(g_ref, z_ref, hw_ref, as_ref, ad_ref, amax_ref, gatb_ref,
              batch_ref, outW_ref, outb_ref, sums_ref, cnt_ref, fin_ref):
    i = pl.program_id(0)
    a_s = as_ref[...]
    a_d = ad_ref[...]
    amax = amax_ref[0, 0]
    e_self = a_s + a_d
    e_self = jnp.where(e_self >= 0.0, e_self, 0.2 * e_self)
    mh = amax + a_d
    mh = jnp.where(mh >= 0.0, mh, 0.2 * mh)
    p_self = jnp.exp(e_self - mh)
    z = z_ref[0] + z_ref[1] + p_self
    hw = hw_ref[...]
    gsum = g_ref[0] + g_ref[1] + p_self * hw
    g = jnp.maximum(gsum / jnp.maximum(z, 1e-16) + gatb_ref[...], 0.0)
    b = batch_ref[0]
    oh = (lax.broadcasted_iota(_i32, (NG, RB), 0) == b).astype(_f32)
    ps = jnp.dot(oh, g, preferred_element_type=_f32)
    pc = jnp.dot(oh, jnp.ones((RB, D), _f32), preferred_element_type=_f32)

    @pl.when(i == 0)
    def _():
        sums_ref[...] = ps
        cnt_ref[...] = pc

    @pl.when(i > 0)
    def _():
        sums_ref[...] = sums_ref[...] + ps
        cnt_ref[...] = cnt_ref[...] + pc

    @pl.when(i == NB - 1)
    def _():
        pooled = sums_ref[...] / jnp.maximum(cnt_ref[...], 1.0)
        fin_ref[...] = (
            jnp.dot(pooled, outW_ref[...], preferred_element_type=_f32)
            + outb_ref[...]
        )


def _tc3(g_p, z_p, hw, a_s, a_d, amax, gat_b, batch3, out_W, out_b):
    sums, cnt, fin = pl.pallas_call(
        _tc3_body,
        grid=(NB,),
        in_specs=[
            pl.BlockSpec((NC, RB, D), lambda i: (0, i, 0)),
            pl.BlockSpec((NC, RB, 1), lambda i: (0, i, 0)),
            pl.BlockSpec((RB, D), lambda i: (i, 0)),
            pl.BlockSpec((RB, 1), lambda i: (i, 0)),
            pl.BlockSpec((RB, 1), lambda i: (i, 0)),
            pl.BlockSpec(memory_space=pltpu.SMEM),
            pl.BlockSpec((1, D), lambda i: (0, 0)),
            pl.BlockSpec((1, 1, RB), lambda i: (i, 0, 0)),
            pl.BlockSpec((D, D), lambda i: (0, 0)),
            pl.BlockSpec((1, D), lambda i: (0, 0)),
        ],
        out_specs=[
            pl.BlockSpec((NG, D), lambda i: (0, 0)),
            pl.BlockSpec((NG, D), lambda i: (0, 0)),
            pl.BlockSpec((NG, D), lambda i: (0, 0)),
        ],
        out_shape=[
            jax.ShapeDtypeStruct((NG, D), _f32),
            jax.ShapeDtypeStruct((NG, D), _f32),
            jax.ShapeDtypeStruct((NG, D), _f32),
        ],
    )(g_p, z_p, hw, a_s, a_d, amax, gat_b, batch3, out_W, out_b)
    return fin


def kernel(x, edge_index, batch, gcn_W, gcn_b, gat_W, att_src, att_dst,
           gat_b, out_W, out_b):
    e_r4 = edge_index.reshape(2, NW, NBLK, BLK, C)
    srcr4, dstr4 = e_r4[0], e_r4[1]
    e_r2 = edge_index.reshape(2, NW, NCH, C)
    srcr2, dstr2 = e_r2[0], e_r2[1]
    z128 = jnp.zeros((NP, D), _f32)
    zN = jnp.zeros((NP,), _f32)
    x_pad = jnp.pad(x, ((0, NP - N), (0, 0)))
    batch_pad = jnp.pad(batch, (0, NP - N), constant_values=NG)

    o1 = jnp.ones((C,), _f32)
    deg_p = _sc_deg(dstr2, zN, o1)
    y, dinv = _tc1(x_pad, gcn_W, deg_p.reshape(NC, NP, 1))
    acc_p = _sc_spmm(y, srcr2, dstr2, z128)
    hw, a_s, a_d, amax = _tc2(
        acc_p, y, dinv,
        gcn_b.reshape(1, D), gat_W,
        att_src.reshape(D, 1), att_dst.reshape(D, 1),
    )
    amax16 = jnp.broadcast_to(amax.reshape(1), (L,))
    g_p, z_p = _sc_gat(
        hw, a_s.reshape(NP), a_d.reshape(NP), amax16, srcr4, dstr4, z128, zN
    )
    batch3 = batch_pad.reshape(NB, 1, RB)
    return _tc3(
        g_p, z_p.reshape(NC, NP, 1), hw, a_s, a_d, amax,
        gat_b.reshape(1, D), batch3, out_W, out_b.reshape(1, D),
    )


# speedup vs baseline: 41.3257x; 1.8033x over previous
import jax
import jax.numpy as jnp
from jax import lax
from jax.experimental import pallas as pl
from jax.experimental.pallas import tpu as pltpu
from jax.experimental.pallas import tpu_sc as plsc

N = 10000
NP = 10240
E = 320000
D = 128
NG = 16

NC = 2
NS = 16
L = 16
NW = NC * NS
EPT = E // NW
C = 80
NCH = EPT // C
BLK = 25
NBLK = NCH // BLK
RPT = NP // NS

RB = 1024
NB = NP // RB

_f32 = jnp.float32
_i32 = jnp.int32


def _mesh():
    return plsc.VectorSubcoreMesh(
        core_axis_name="c", subcore_axis_name="s", num_cores=NC, num_subcores=NS
    )


def _sc_deg_body(dstr, zN, o1, out, dst_v, obuf, deg_sp):
    c = lax.axis_index("c")
    s = lax.axis_index("s")
    wid = s * NC + c
    rows = pl.ds(s * RPT, RPT)
    pltpu.sync_copy(dstr.at[wid], dst_v)
    pltpu.sync_copy(zN.at[rows], deg_sp.at[rows])
    pltpu.sync_copy(o1, obuf)
    plsc.subcore_barrier()

    def body(j, carry):
        pltpu.sync_copy(obuf, deg_sp.at[dst_v.at[j]], add=True)
        return carry

    lax.fori_loop(0, NCH, body, 0)
    plsc.subcore_barrier()
    pltpu.sync_copy(deg_sp.at[rows], out.at[c, rows])


def _sc_deg(dstr, zN, o1):
    kern = pl.kernel(
        _sc_deg_body,
        out_type=jax.ShapeDtypeStruct((NC, NP), _f32),
        mesh=_mesh(),
        scratch_types=[
            pltpu.VMEM((NCH, C), _i32),
            pltpu.VMEM((C,), _f32),
            pltpu.VMEM_SHARED((NP,), _f32),
        ],
    )
    return kern(dstr, zN, o1)


def _sc_spmm_body(y, srcr, dstr, z128, out, src_v, dst_v, buf0, buf1,
                  sem0, sem1, acc_sp):
    c = lax.axis_index("c")
    s = lax.axis_index("s")
    wid = s * NC + c
    rows = pl.ds(s * RPT, RPT)
    pltpu.sync_copy(z128.at[rows], acc_sp.at[rows])
    plsc.subcore_barrier()

    def blk_body(b, carry):
        pltpu.sync_copy(srcr.at[wid, b], src_v)
        pltpu.sync_copy(dstr.at[wid, b], dst_v)
        pltpu.async_copy(y.at[src_v.at[0]], buf0, sem0)

        def pair(i, carry2):
            a = 2 * i + 1
            pltpu.async_copy(y.at[src_v.at[a]], buf1, sem1)
            pltpu.make_async_copy(y.at[src_v.at[a - 1]], buf0, sem0).wait()
            pltpu.sync_copy(buf0, acc_sp.at[dst_v.at[a - 1]], add=True)
            pltpu.async_copy(y.at[src_v.at[a + 1]], buf0, sem0)
            pltpu.make_async_copy(y.at[src_v.at[a]], buf1, sem1).wait()
            pltpu.sync_copy(buf1, acc_sp.at[dst_v.at[a]], add=True)
            return carry2

        lax.fori_loop(0, (BLK - 1) // 2, pair, 0)
        pltpu.make_async_copy(y.at[src_v.at[BLK - 1]], buf0, sem0).wait()
        pltpu.sync_copy(buf0, acc_sp.at[dst_v.at[BLK - 1]], add=True)
        return carry

    lax.fori_loop(0, NBLK, blk_body, 0)
    plsc.subcore_barrier()
    pltpu.sync_copy(acc_sp.at[rows], out.at[c, rows])


def _sc_spmm(y, srcr, dstr, z128):
    kern = pl.kernel(
        _sc_spmm_body,
        out_type=jax.ShapeDtypeStruct((NC, NP, D), _f32),
        mesh=_mesh(),
        scratch_types=[
            pltpu.VMEM((BLK, C), _i32),
            pltpu.VMEM((BLK, C), _i32),
            pltpu.VMEM((C, D), _f32),
            pltpu.VMEM((C, D), _f32),
            pltpu.SemaphoreType.DMA,
            pltpu.SemaphoreType.DMA,
            pltpu.VMEM_SHARED((NP, D), _f32),
        ],
    )
    return kern(y, srcr, dstr, z128)


def _sc_gat_body(hw, a_s, a_d, amax, srcr, dstr, z128, zN, g_out, z_out,
                 src_v, dst_v, asb0, adb0, asb1, adb1, amx, pbuf, buf0, buf1,
                 sem0, sem1, g_sp, z1_sp):
    c = lax.axis_index("c")
    s = lax.axis_index("s")
    wid = s * NC + c
    rows = pl.ds(s * RPT, RPT)
    pltpu.sync_copy(amax, amx)
    pltpu.sync_copy(z128.at[rows], g_sp.at[rows])
    pltpu.sync_copy(zN.at[rows], z1_sp.at[rows])
    plsc.subcore_barrier()

    amx_v0 = amx[pl.ds(0, L)]

    def fire(k, buf, asb, adb, sem):
        pltpu.async_copy(hw.at[src_v.at[k]], buf, sem)
        pltpu.async_copy(a_s.at[src_v.at[k]], asb, sem)
        pltpu.async_copy(a_d.at[dst_v.at[k]], adb, sem)

    def wait(k, buf, asb, adb, sem):
        pltpu.make_async_copy(hw.at[src_v.at[k]], buf, sem).wait()
        pltpu.make_async_copy(a_s.at[src_v.at[k]], asb, sem).wait()
        pltpu.make_async_copy(a_d.at[dst_v.at[k]], adb, sem).wait()

    def process(k, buf, asb, adb, pbuf):
        def scale(g, carry3):
            sl = pl.ds(g * L, L)
            av = asb[sl]
            bv = adb[sl]
            e = av + bv
            e = jnp.where(e >= 0.0, e, 0.2 * e)
            mh = amx_v0 + bv
            mh = jnp.where(mh >= 0.0, mh, 0.2 * mh)
            p = jnp.exp(e - mh)
            pbuf[sl] = p
            for r in range(L):
                lrow = g * L + r
                ps = p[r]
                for v8 in range(D // L):
                    fsl = pl.ds(v8 * L, L)
                    buf[lrow, fsl] = buf[lrow, fsl] * ps
            return carry3

        lax.fori_loop(0, C // L, scale, 0)
        pltpu.sync_copy(buf, g_sp.at[dst_v.at[k]], add=True)
        pltpu.sync_copy(pbuf, z1_sp.at[dst_v.at[k]], add=True)

    def blk_body(b, carry):
        pltpu.sync_copy(srcr.at[wid, b], src_v)
        pltpu.sync_copy(dstr.at[wid, b], dst_v)
        fire(0, buf0, asb0, adb0, sem0)

        def pair(i, carry2):
            a = 2 * i + 1
            fire(a, buf1, asb1, adb1, sem1)
            wait(a - 1, buf0, asb0, adb0, sem0)
            process(a - 1, buf0, asb0, adb0, pbuf)
            fire(a + 1, buf0, asb0, adb0, sem0)
            wait(a, buf1, asb1, adb1, sem1)
            process(a, buf1, asb1, adb1, pbuf)
            return carry2

        lax.fori_loop(0, (BLK - 1) // 2, pair, 0)
        wait(BLK - 1, buf0, asb0, adb0, sem0)
        process(BLK - 1, buf0, asb0, adb0, pbuf)
        return carry

    lax.fori_loop(0, NBLK, blk_body, 0)
    plsc.subcore_barrier()
    pltpu.sync_copy(g_sp.at[rows], g_out.at[c, rows])
    pltpu.sync_copy(z1_sp.at[rows], z_out.at[c, rows])


def _sc_gat(hw, a_s, a_d, amax, srcr, dstr, z128, zN):
    kern = pl.kernel(
        _sc_gat_body,
        out_type=(
            jax.ShapeDtypeStruct((NC, NP, D), _f32),
            jax.ShapeDtypeStruct((NC, NP), _f32),
        ),
        mesh=_mesh(),
        scratch_types=[
            pltpu.VMEM((BLK, C), _i32),
            pltpu.VMEM((BLK, C), _i32),
            pltpu.VMEM((C,), _f32),
            pltpu.VMEM((C,), _f32),
            pltpu.VMEM((C,), _f32),
            pltpu.VMEM((C,), _f32),
            pltpu.VMEM((L,), _f32),
            pltpu.VMEM((C,), _f32),
            pltpu.VMEM((C, D), _f32),
            pltpu.VMEM((C, D), _f32),
            pltpu.SemaphoreType.DMA,
            pltpu.SemaphoreType.DMA,
            pltpu.VMEM_SHARED((NP, D), _f32),
            pltpu.VMEM_SHARED((NP,), _f32),
        ],
    )
    return kern(hw, a_s, a_d, amax, srcr, dstr, z128, zN)


def _tc1_body(x_ref, w_ref, deg_ref, y_ref, dinv_ref):
    deg = deg_ref[0] + deg_ref[1] + 1.0
    dinv = lax.rsqrt(deg)
    xw = jnp.dot(x_ref[...], w_ref[...], preferred_element_type=_f32)
    y_ref[...] = xw * dinv
    dinv_ref[...] = dinv


def _tc1(x, gcn_W, deg_p):
    return pl.pallas_call(
        _tc1_body,
        grid=(NB,),
        in_specs=[
            pl.BlockSpec((RB, D), lambda i: (i, 0)),
            pl.BlockSpec((D, D), lambda i: (0, 0)),
            pl.BlockSpec((NC, RB, 1), lambda i: (0, i, 0)),
        ],
        out_specs=[
            pl.BlockSpec((RB, D), lambda i: (i, 0)),
            pl.BlockSpec((RB, 1), lambda i: (i, 0)),
        ],
        out_shape=[
            jax.ShapeDtypeStruct((NP, D), _f32),
            jax.ShapeDtypeStruct((NP, 1), _f32),
        ],
    )(x, gcn_W, deg_p)


def _tc2_body(acc_ref, y_ref, dinv_ref, gcnb_ref, gatW_ref, atts_ref, attd_ref,
              hw_ref, as_ref, ad_ref, amax_ref):
    i = pl.program_id(0)
    acc = acc_ref[0] + acc_ref[1] + y_ref[...]
    h = jnp.maximum(acc * dinv_ref[...] + gcnb_ref[...], 0.0)
    hw = jnp.dot(h, gatW_ref[...], preferred_element_type=_f32)
    hw_ref[...] = hw
    a_s = jnp.dot(hw, atts_ref[...], preferred_element_type=_f32)
    a_d = jnp.dot(hw, attd_ref[...], preferred_element_type=_f32)
    as_ref[...] = a_s
    ad_ref[...] = a_d
    m = jnp.max(a_s)

    @pl.when(i == 0)
    def _():
        amax_ref[0, 0] = m

    @pl.when(i > 0)
    def _():
        amax_ref[0, 0] = jnp.maximum(amax_ref[0, 0], m)


def _tc2(acc_p, y, dinv, gcn_b, gat_W, atts, attd):
    return pl.pallas_call(
        _tc2_body,
        grid=(NB,),
        in_specs=[
            pl.BlockSpec((NC, RB, D), lambda i: (0, i, 0)),
            pl.BlockSpec((RB, D), lambda i: (i, 0)),
            pl.BlockSpec((RB, 1), lambda i: (i, 0)),
            pl.BlockSpec((1, D), lambda i: (0, 0)),
            pl.BlockSpec((D, D), lambda i: (0, 0)),
            pl.BlockSpec((D, 1), lambda i: (0, 0)),
            pl.BlockSpec((D, 1), lambda i: (0, 0)),
        ],
        out_specs=[
            pl.BlockSpec((RB, D), lambda i: (i, 0)),
            pl.BlockSpec((RB, 1), lambda i: (i, 0)),
            pl.BlockSpec((RB, 1), lambda i: (i, 0)),
            pl.BlockSpec(memory_space=pltpu.SMEM),
        ],
        out_shape=[
            jax.ShapeDtypeStruct((NP, D), _f32),
            jax.ShapeDtypeStruct((NP, 1), _f32),
            jax.ShapeDtypeStruct((NP, 1), _f32),
            jax.ShapeDtypeStruct((1, 1), _f32),
        ],
    )(acc_p, y, dinv, gcn_b, gat_W, atts, attd)


def _tc3_body(g_ref, z_ref, hw_ref, as_ref, ad_ref, amax_ref, gatb_ref,
              batch_ref, outW_ref, outb_ref, sums_ref, cnt_ref, fin_ref):
    i = pl.program_id(0)
    a_s = as_ref[...]
    a_d = ad_ref[...]
    amax = amax_ref[0, 0]
    e_self = a_s + a_d
    e_self = jnp.where(e_self >= 0.0, e_self, 0.2 * e_self)
    mh = amax + a_d
    mh = jnp.where(mh >= 0.0, mh, 0.2 * mh)
    p_self = jnp.exp(e_self - mh)
    z = z_ref[0] + z_ref[1] + p_self
    hw = hw_ref[...]
    gsum = g_ref[0] + g_ref[1] + p_self * hw
    g = jnp.maximum(gsum / jnp.maximum(z, 1e-16) + gatb_ref[...], 0.0)
    b = batch_ref[0]
    oh = (lax.broadcasted_iota(_i32, (NG, RB), 0) == b).astype(_f32)
    ps = jnp.dot(oh, g, preferred_element_type=_f32)
    pc = jnp.dot(oh, jnp.ones((RB, D), _f32), preferred_element_type=_f32)

    @pl.when(i == 0)
    def _():
        sums_ref[...] = ps
        cnt_ref[...] = pc

    @pl.when(i > 0)
    def _():
        sums_ref[...] = sums_ref[...] + ps
        cnt_ref[...] = cnt_ref[...] + pc

    @pl.when(i == NB - 1)
    def _():
        pooled = sums_ref[...] / jnp.maximum(cnt_ref[...], 1.0)
        fin_ref[...] = (
            jnp.dot(pooled, outW_ref[...], preferred_element_type=_f32)
            + outb_ref[...]
        )


def _tc3(g_p, z_p, hw, a_s, a_d, amax, gat_b, batch3, out_W, out_b):
    sums, cnt, fin = pl.pallas_call(
        _tc3_body,
        grid=(NB,),
        in_specs=[
            pl.BlockSpec((NC, RB, D), lambda i: (0, i, 0)),
            pl.BlockSpec((NC, RB, 1), lambda i: (0, i, 0)),
            pl.BlockSpec((RB, D), lambda i: (i, 0)),
            pl.BlockSpec((RB, 1), lambda i: (i, 0)),
            pl.BlockSpec((RB, 1), lambda i: (i, 0)),
            pl.BlockSpec(memory_space=pltpu.SMEM),
            pl.BlockSpec((1, D), lambda i: (0, 0)),
            pl.BlockSpec((1, 1, RB), lambda i: (i, 0, 0)),
            pl.BlockSpec((D, D), lambda i: (0, 0)),
            pl.BlockSpec((1, D), lambda i: (0, 0)),
        ],
        out_specs=[
            pl.BlockSpec((NG, D), lambda i: (0, 0)),
            pl.BlockSpec((NG, D), lambda i: (0, 0)),
            pl.BlockSpec((NG, D), lambda i: (0, 0)),
        ],
        out_shape=[
            jax.ShapeDtypeStruct((NG, D), _f32),
            jax.ShapeDtypeStruct((NG, D), _f32),
            jax.ShapeDtypeStruct((NG, D), _f32),
        ],
    )(g_p, z_p, hw, a_s, a_d, amax, gat_b, batch3, out_W, out_b)
    return fin


def kernel(x, edge_index, batch, gcn_W, gcn_b, gat_W, att_src, att_dst,
           gat_b, out_W, out_b):
    e_r4 = edge_index.reshape(2, NW, NBLK, BLK, C)
    srcr4, dstr4 = e_r4[0], e_r4[1]
    e_r2 = edge_index.reshape(2, NW, NCH, C)
    srcr2, dstr2 = e_r2[0], e_r2[1]
    z128 = jnp.zeros((NP, D), _f32)
    zN = jnp.zeros((NP,), _f32)
    x_pad = jnp.pad(x, ((0, NP - N), (0, 0)))
    batch_pad = jnp.pad(batch, (0, NP - N), constant_values=NG)

    o1 = jnp.ones((C,), _f32)
    deg_p = _sc_deg(dstr2, zN, o1)
    y, dinv = _tc1(x_pad, gcn_W, deg_p.reshape(NC, NP, 1))
    acc_p = _sc_spmm(y, srcr4, dstr4, z128)
    hw, a_s, a_d, amax = _tc2(
        acc_p, y, dinv,
        gcn_b.reshape(1, D), gat_W,
        att_src.reshape(D, 1), att_dst.reshape(D, 1),
    )
    amax16 = jnp.broadcast_to(amax.reshape(1), (L,))
    g_p, z_p = _sc_gat(
        hw, a_s.reshape(NP), a_d.reshape(NP), amax16, srcr4, dstr4, z128, zN
    )
    batch3 = batch_pad.reshape(NB, 1, RB)
    return _tc3(
        g_p, z_p.reshape(NC, NP, 1), hw, a_s, a_d, amax,
        gat_b.reshape(1, D), batch3, out_W, out_b.reshape(1, D),
    )
